# Initial kernel scaffold; baseline (speedup 1.0000x reference)
import functools

import jax
import jax.numpy as jnp
from jax import lax
from jax.experimental import pallas as pl
from jax.experimental.pallas import tpu as pltpu
from jax.experimental.pallas import tpu_sc as plsc

N = 10000
C = 128
H = 256
K = 128
NT = 16
NC = 2
NPAD = NT * 640
BM = 1000
f32 = jnp.float32
i32 = jnp.int32

_mesh = plsc.VectorSubcoreMesh(core_axis_name="c", subcore_axis_name="s")


def _fast_rsqrt(x):
    bi = lax.bitcast_convert_type(x, i32)
    bi = 0x5F3759DF - lax.shift_right_arithmetic(bi, 1)
    y = lax.bitcast_convert_type(bi, f32)
    for _ in range(3):
        y = y * (1.5 - 0.5 * x * y * y)
    return y


def _prep_body(src_h, dst_h, w_h, norm_h, degS, idxv, didxv, valv, degv, disv, normv):
    c = lax.axis_index("c")
    s = lax.axis_index("s")
    E2 = src_h.shape[0]
    npt = E2 // NT
    ncw = E2 // (NT * NC)

    zero16 = jnp.zeros((16,), f32)
    for j in range(8):
        valv[pl.ds(j * 16, 16)] = zero16
    for r in range(5):
        pltpu.sync_copy(valv, degS.at[pl.ds(s * 640 + r * K, K)])
    plsc.subcore_barrier()

    def deg_chunk(ci, carry):
        base = s * npt + ci * K
        pltpu.sync_copy(dst_h.at[pl.ds(base, K)], idxv)
        pltpu.sync_copy(w_h.at[pl.ds(base, K)], valv)
        pltpu.sync_copy(valv, degS.at[idxv], add=True)
        return carry

    lax.fori_loop(0, npt // K, deg_chunk, 0)
    plsc.subcore_barrier()

    pltpu.sync_copy(degS.at[pl.ds(s * 640, 640)], degv)
    for j in range(40):
        sl = pl.ds(j * 16, 16)
        degv[sl] = _fast_rsqrt(degv[sl])
    pltpu.sync_copy(degv, degS.at[pl.ds(s * 640, 640)])
    plsc.subcore_barrier()

    pltpu.sync_copy(degS.at[pl.ds(0, N)], disv)
    wid = c * NT + s

    def norm_chunk(ci, carry):
        base = wid * ncw + ci * K
        pltpu.sync_copy(src_h.at[pl.ds(base, K)], idxv)
        pltpu.sync_copy(dst_h.at[pl.ds(base, K)], didxv)
        pltpu.sync_copy(w_h.at[pl.ds(base, K)], valv)
        for k in range(8):
            sl = pl.ds(k * 16, 16)
            nm = plsc.load_gather(disv, [idxv[sl]]) * valv[sl] * plsc.load_gather(disv, [didxv[sl]])
            normv[sl] = nm
        pltpu.sync_copy(normv, norm_h.at[pl.ds(base, K)])
        return carry

    lax.fori_loop(0, ncw // K, norm_chunk, 0)


def _conv_body(xw_h, src_h, dst_h, norm_h, b_h, out_h, accS, idxv, dstv, normv, rows, biasv):
    c = lax.axis_index("c")
    s = lax.axis_index("s")
    E2 = src_h.shape[0]
    npt = E2 // NT

    zero16 = jnp.zeros((16,), f32)

    def zrow(r, carry):
        for j in range(8):
            rows[r, pl.ds(j * 16, 16)] = zero16
        return carry

    lax.fori_loop(0, K, zrow, 0)
    for r in range(5):
        pltpu.sync_copy(rows, accS.at[pl.ds(s * 640 + r * K, K)])
    plsc.subcore_barrier()
    pltpu.sync_copy(b_h.at[pl.ds(c * 128, 128)], biasv)

    def chunk(ci, carry):
        base = s * npt + ci * K
        pltpu.sync_copy(src_h.at[pl.ds(base, K)], idxv)
        coff = c * N
        for j in range(8):
            sl = pl.ds(j * 16, 16)
            idxv[sl] = idxv[sl] + coff
        pltpu.sync_copy(dst_h.at[pl.ds(base, K)], dstv)
        pltpu.sync_copy(norm_h.at[pl.ds(base, K)], normv)
        pltpu.sync_copy(xw_h.at[idxv], rows)

        def scale16(k16, carry2):
            for l in range(16):
                k = k16 * 16 + l
                nb = plsc.load_gather(normv, [jnp.zeros((16,), i32) + k])
                for j in range(8):
                    sl = pl.ds(j * 16, 16)
                    rows[k, sl] = rows[k, sl] * nb
            return carry2

        lax.fori_loop(0, K // 16, scale16, 0)
        pltpu.sync_copy(rows, accS.at[dstv], add=True)
        return carry

    lax.fori_loop(0, npt // K, chunk, 0)
    plsc.subcore_barrier()

    bvecs = [biasv[pl.ds(j * 16, 16)] for j in range(8)]
    for piece in range(5):
        r0 = s * 625 + piece * 125
        pltpu.sync_copy(accS.at[pl.ds(r0, 125)], rows.at[pl.ds(0, 125)])

        def ep(r, carry):
            for j in range(8):
                sl = pl.ds(j * 16, 16)
                rows[r, sl] = jnp.maximum(rows[r, sl] + bvecs[j], 0.0)
            return carry

        lax.fori_loop(0, 125, ep, 0)
        pltpu.sync_copy(rows.at[pl.ds(0, 125)], out_h.at[pl.ds(c * N + r0, 125)])


def _make_prep(E2):
    return functools.partial(
        pl.kernel,
        out_type=jax.ShapeDtypeStruct((E2,), f32),
        mesh=_mesh,
        scratch_types=[
            pltpu.VMEM_SHARED((NPAD,), f32),
            pltpu.VMEM((K,), i32),
            pltpu.VMEM((K,), i32),
            pltpu.VMEM((K,), f32),
            pltpu.VMEM((640,), f32),
            pltpu.VMEM((N,), f32),
            pltpu.VMEM((K,), f32),
        ],
    )(_prep_body)


def _make_conv():
    return functools.partial(
        pl.kernel,
        out_type=jax.ShapeDtypeStruct((NC * N, 128), f32),
        mesh=_mesh,
        scratch_types=[
            pltpu.VMEM_SHARED((NPAD, 128), f32),
            pltpu.VMEM((K,), i32),
            pltpu.VMEM((K,), i32),
            pltpu.VMEM((K,), f32),
            pltpu.VMEM((K, 128), f32),
            pltpu.VMEM((128,), f32),
        ],
    )(_conv_body)


def _mm1_body(x_ref, w_ref, o_ref):
    o_ref[...] = jnp.dot(x_ref[...], w_ref[...], preferred_element_type=f32)


def _mm2_body(xa_ref, xb_ref, w_ref, o_ref):
    o_ref[...] = (jnp.dot(xa_ref[...], w_ref[0:128], preferred_element_type=f32)
                  + jnp.dot(xb_ref[...], w_ref[128:256], preferred_element_type=f32))


def _lstm_body(x1a, x1b, x2a, x2b, x_ref, wt1, bb1, wt2, bb2, o_ref):
    g1 = (jnp.dot(x1a[...], wt1[0:128], preferred_element_type=f32)
          + jnp.dot(x1b[...], wt1[128:256], preferred_element_type=f32)
          + jnp.dot(x2a[...], wt1[256:384], preferred_element_type=f32)
          + jnp.dot(x2b[...], wt1[384:512], preferred_element_type=f32)) + bb1[...]
    gi = g1[:, 0:H]
    gg = g1[:, H:2 * H]
    go = g1[:, 2 * H:3 * H]
    cell = jax.nn.sigmoid(gi) * jnp.tanh(gg)
    h1 = jax.nn.sigmoid(go) * jnp.tanh(cell)
    g2 = jnp.dot(h1, wt2[...], preferred_element_type=f32) + bb2[...]
    gi2 = g2[:, 0:H]
    gg2 = g2[:, H:2 * H]
    go2 = g2[:, 2 * H:3 * H]
    cell2 = jax.nn.sigmoid(gi2) * jnp.tanh(gg2)
    h2 = jax.nn.sigmoid(go2) * jnp.tanh(cell2)
    o_ref[:, 0:H] = h1
    o_ref[:, H:2 * H] = h2
    o_ref[:, 2 * H:2 * H + C] = x_ref[...]


def kernel(X, edge_index, edge_weight, W1, b1, W2, b2,
           W_ih1, W_hh1, b_ih1, b_hh1, W_ih2, W_hh2, b_ih2, b_hh2):
    E = edge_weight.shape[0]
    loop = jnp.arange(N, dtype=edge_index.dtype)
    src = jnp.concatenate([edge_index[0], loop])
    dst = jnp.concatenate([edge_index[1], loop])
    w = jnp.concatenate([edge_weight, jnp.ones((N,), f32)])
    E2 = ((E + N + NT * NC * K - 1) // (NT * NC * K)) * (NT * NC * K)
    pad = E2 - (E + N)
    if pad:
        src = jnp.concatenate([src, jnp.zeros((pad,), src.dtype)])
        dst = jnp.concatenate([dst, jnp.zeros((pad,), dst.dtype)])
        w = jnp.concatenate([w, jnp.zeros((pad,), f32)])

    norm = _make_prep(E2)(src, dst, w)

    nblk = N // BM
    xw1 = pl.pallas_call(
        _mm1_body,
        grid=(nblk, NC),
        in_specs=[pl.BlockSpec((BM, C), lambda i, cc: (i, 0)),
                  pl.BlockSpec((C, 128), lambda i, cc: (0, cc))],
        out_specs=pl.BlockSpec((BM, 128), lambda i, cc: (cc * nblk + i, 0)),
        out_shape=jax.ShapeDtypeStruct((NC * N, 128), f32),
    )(X, W1)

    conv = _make_conv()
    x1 = conv(xw1, src, dst, norm, b1)

    xw2 = pl.pallas_call(
        _mm2_body,
        grid=(nblk, NC),
        in_specs=[pl.BlockSpec((BM, 128), lambda i, cc: (i, 0)),
                  pl.BlockSpec((BM, 128), lambda i, cc: (nblk + i, 0)),
                  pl.BlockSpec((H, 128), lambda i, cc: (0, cc))],
        out_specs=pl.BlockSpec((BM, 128), lambda i, cc: (cc * nblk + i, 0)),
        out_shape=jax.ShapeDtypeStruct((NC * N, 128), f32),
    )(x1, x1, W2)

    x2 = conv(xw2, src, dst, norm, b2)

    bb1 = b_ih1 + b_hh1
    Wt1 = jnp.concatenate([W_ih1[0:H], W_ih1[2 * H:4 * H]], axis=0).T
    bb1 = jnp.concatenate([bb1[0:H], bb1[2 * H:4 * H]]).reshape(1, 3 * H)
    bb2 = b_ih2 + b_hh2
    Wt2 = jnp.concatenate([W_ih2[0:H], W_ih2[2 * H:4 * H]], axis=0).T
    bb2 = jnp.concatenate([bb2[0:H], bb2[2 * H:4 * H]]).reshape(1, 3 * H)

    out = pl.pallas_call(
        _lstm_body,
        grid=(nblk,),
        in_specs=[pl.BlockSpec((BM, 128), lambda i: (i, 0)),
                  pl.BlockSpec((BM, 128), lambda i: (nblk + i, 0)),
                  pl.BlockSpec((BM, 128), lambda i: (i, 0)),
                  pl.BlockSpec((BM, 128), lambda i: (nblk + i, 0)),
                  pl.BlockSpec((BM, C), lambda i: (i, 0)),
                  pl.BlockSpec((2 * H, 3 * H), lambda i: (0, 0)),
                  pl.BlockSpec((1, 3 * H), lambda i: (0, 0)),
                  pl.BlockSpec((H, 3 * H), lambda i: (0, 0)),
                  pl.BlockSpec((1, 3 * H), lambda i: (0, 0)),
                  ],
        out_specs=pl.BlockSpec((BM, 2 * H + C), lambda i: (i, 0)),
        out_shape=jax.ShapeDtypeStruct((N, 2 * H + C), f32),
    )(x1, x1, x2, x2, X, Wt1, bb1, Wt2, bb2)
    return out


# speedup vs baseline: 6.0566x; 6.0566x over previous
import functools

import jax
import jax.numpy as jnp
from jax import lax
from jax.experimental import pallas as pl
from jax.experimental.pallas import tpu as pltpu
from jax.experimental.pallas import tpu_sc as plsc

N = 10000
C = 128
H = 256
K = 128
NT = 16
NC = 2
NPAD = NT * 640
BM = 1000
f32 = jnp.float32
i32 = jnp.int32

_mesh = plsc.VectorSubcoreMesh(core_axis_name="c", subcore_axis_name="s")


def _fast_rsqrt(x):
    bi = lax.bitcast_convert_type(x, i32)
    bi = 0x5F3759DF - lax.shift_right_arithmetic(bi, 1)
    y = lax.bitcast_convert_type(bi, f32)
    for _ in range(3):
        y = y * (1.5 - 0.5 * x * y * y)
    return y


def _prep_body(src_h, dst_h, w_h, norm_h, degS, idxv, didxv, valv, degv, disv, normv):
    c = lax.axis_index("c")
    s = lax.axis_index("s")
    E2 = src_h.shape[0]
    npt = E2 // NT
    ncw = E2 // (NT * NC)

    zero16 = jnp.zeros((16,), f32)
    for j in range(8):
        valv[pl.ds(j * 16, 16)] = zero16
    for r in range(5):
        pltpu.sync_copy(valv, degS.at[pl.ds(s * 640 + r * K, K)])
    plsc.subcore_barrier()

    def deg_chunk(ci, carry):
        base = s * npt + ci * K
        pltpu.sync_copy(dst_h.at[pl.ds(base, K)], idxv)
        pltpu.sync_copy(w_h.at[pl.ds(base, K)], valv)
        pltpu.sync_copy(valv, degS.at[idxv], add=True)
        return carry

    lax.fori_loop(0, npt // K, deg_chunk, 0)
    plsc.subcore_barrier()

    pltpu.sync_copy(degS.at[pl.ds(s * 640, 640)], degv)
    for j in range(40):
        sl = pl.ds(j * 16, 16)
        degv[sl] = _fast_rsqrt(degv[sl])
    pltpu.sync_copy(degv, degS.at[pl.ds(s * 640, 640)])
    plsc.subcore_barrier()

    pltpu.sync_copy(degS.at[pl.ds(0, N)], disv)
    wid = c * NT + s

    def norm_chunk(ci, carry):
        base = wid * ncw + ci * K
        pltpu.sync_copy(src_h.at[pl.ds(base, K)], idxv)
        pltpu.sync_copy(dst_h.at[pl.ds(base, K)], didxv)
        pltpu.sync_copy(w_h.at[pl.ds(base, K)], valv)
        for k in range(8):
            sl = pl.ds(k * 16, 16)
            nm = plsc.load_gather(disv, [idxv[sl]]) * valv[sl] * plsc.load_gather(disv, [didxv[sl]])
            normv[sl] = nm
        pltpu.sync_copy(normv, norm_h.at[pl.ds(base, K)])
        return carry

    lax.fori_loop(0, ncw // K, norm_chunk, 0)


def _conv_body(xw_h, src_h, dst_h, norm_h, b_h, out_h, accS, idxv, dstv, normv, rows, biasv):
    c = lax.axis_index("c")
    s = lax.axis_index("s")
    E2 = src_h.shape[0]
    npt = E2 // NT

    zero16 = jnp.zeros((16,), f32)

    def zrow(r, carry):
        for j in range(8):
            rows[r, pl.ds(j * 16, 16)] = zero16
        return carry

    lax.fori_loop(0, K, zrow, 0)
    for r in range(5):
        pltpu.sync_copy(rows, accS.at[pl.ds(s * 640 + r * K, K)])
    plsc.subcore_barrier()
    pltpu.sync_copy(b_h.at[pl.ds(c * 128, 128)], biasv)

    def chunk(ci, carry):
        base = s * npt + ci * K
        pltpu.sync_copy(src_h.at[pl.ds(base, K)], idxv)
        coff = c * N
        for j in range(8):
            sl = pl.ds(j * 16, 16)
            idxv[sl] = idxv[sl] + coff
        pltpu.sync_copy(dst_h.at[pl.ds(base, K)], dstv)
        pltpu.sync_copy(norm_h.at[pl.ds(base, K)], normv)
        pltpu.sync_copy(xw_h.at[idxv], rows)

        def scale16(k16, carry2):
            for l in range(16):
                k = k16 * 16 + l
                nb = plsc.load_gather(normv, [jnp.zeros((16,), i32) + k])
                for j in range(8):
                    sl = pl.ds(j * 16, 16)
                    rows[k, sl] = rows[k, sl] * nb
            return carry2

        lax.fori_loop(0, K // 16, scale16, 0)
        pltpu.sync_copy(rows, accS.at[dstv], add=True)
        return carry

    lax.fori_loop(0, npt // K, chunk, 0)
    plsc.subcore_barrier()

    bvecs = [biasv[pl.ds(j * 16, 16)] for j in range(8)]

    def ep(r, carry):
        for j in range(8):
            sl = pl.ds(j * 16, 16)
            rows[r, sl] = jnp.maximum(rows[r, sl] + bvecs[j], 0.0)
        return carry

    def do_piece(r0, ln):
        pltpu.sync_copy(accS.at[pl.ds(r0, ln)], rows.at[pl.ds(0, ln)])
        lax.fori_loop(0, ln, ep, 0)
        pltpu.sync_copy(rows.at[pl.ds(0, ln)], out_h.at[pl.ds(c * N + r0, ln)])

    base = s * 624
    for off, ln in ((0, K), (K, K), (2 * K, K), (3 * K, K), (4 * K, 112)):
        do_piece(base + off, ln)

    @pl.when(s == NT - 1)
    def _tail():
        do_piece(9984, 16)


_sc_params = pltpu.CompilerParams(needs_layout_passes=False)


def _make_prep(E2):
    return functools.partial(
        pl.kernel,
        out_type=jax.ShapeDtypeStruct((E2,), f32),
        mesh=_mesh,
        compiler_params=_sc_params,
        scratch_types=[
            pltpu.VMEM_SHARED((NPAD,), f32),
            pltpu.VMEM((K,), i32),
            pltpu.VMEM((K,), i32),
            pltpu.VMEM((K,), f32),
            pltpu.VMEM((640,), f32),
            pltpu.VMEM((N,), f32),
            pltpu.VMEM((K,), f32),
        ],
    )(_prep_body)


def _make_conv():
    return functools.partial(
        pl.kernel,
        out_type=jax.ShapeDtypeStruct((NC * N, 128), f32),
        mesh=_mesh,
        compiler_params=_sc_params,
        scratch_types=[
            pltpu.VMEM_SHARED((NPAD, 128), f32),
            pltpu.VMEM((K,), i32),
            pltpu.VMEM((K,), i32),
            pltpu.VMEM((K,), f32),
            pltpu.VMEM((K, 128), f32),
            pltpu.VMEM((128,), f32),
        ],
    )(_conv_body)


def _mm1_body(x_ref, w_ref, o_ref):
    o_ref[...] = jnp.dot(x_ref[...], w_ref[...], preferred_element_type=f32)


def _mm2_body(xa_ref, xb_ref, w_ref, o_ref):
    o_ref[...] = (jnp.dot(xa_ref[...], w_ref[0:128], preferred_element_type=f32)
                  + jnp.dot(xb_ref[...], w_ref[128:256], preferred_element_type=f32))


def _lstm_body(x1a, x1b, x2a, x2b, x_ref, wt1, bb1, wt2, bb2, o_ref):
    g1 = (jnp.dot(x1a[...], wt1[0:128], preferred_element_type=f32)
          + jnp.dot(x1b[...], wt1[128:256], preferred_element_type=f32)
          + jnp.dot(x2a[...], wt1[256:384], preferred_element_type=f32)
          + jnp.dot(x2b[...], wt1[384:512], preferred_element_type=f32)) + bb1[...]
    gi = g1[:, 0:H]
    gg = g1[:, H:2 * H]
    go = g1[:, 2 * H:3 * H]
    cell = jax.nn.sigmoid(gi) * jnp.tanh(gg)
    h1 = jax.nn.sigmoid(go) * jnp.tanh(cell)
    g2 = jnp.dot(h1, wt2[...], preferred_element_type=f32) + bb2[...]
    gi2 = g2[:, 0:H]
    gg2 = g2[:, H:2 * H]
    go2 = g2[:, 2 * H:3 * H]
    cell2 = jax.nn.sigmoid(gi2) * jnp.tanh(gg2)
    h2 = jax.nn.sigmoid(go2) * jnp.tanh(cell2)
    o_ref[:, 0:H] = h1
    o_ref[:, H:2 * H] = h2
    o_ref[:, 2 * H:2 * H + C] = x_ref[...]


def kernel(X, edge_index, edge_weight, W1, b1, W2, b2,
           W_ih1, W_hh1, b_ih1, b_hh1, W_ih2, W_hh2, b_ih2, b_hh2):
    E = edge_weight.shape[0]
    loop = jnp.arange(N, dtype=edge_index.dtype)
    src = jnp.concatenate([edge_index[0], loop])
    dst = jnp.concatenate([edge_index[1], loop])
    w = jnp.concatenate([edge_weight, jnp.ones((N,), f32)])
    E2 = ((E + N + NT * NC * K - 1) // (NT * NC * K)) * (NT * NC * K)
    pad = E2 - (E + N)
    if pad:
        src = jnp.concatenate([src, jnp.zeros((pad,), src.dtype)])
        dst = jnp.concatenate([dst, jnp.zeros((pad,), dst.dtype)])
        w = jnp.concatenate([w, jnp.zeros((pad,), f32)])

    norm = _make_prep(E2)(src, dst, w)

    nblk = N // BM
    xw1 = pl.pallas_call(
        _mm1_body,
        grid=(nblk, NC),
        in_specs=[pl.BlockSpec((BM, C), lambda i, cc: (i, 0)),
                  pl.BlockSpec((C, 128), lambda i, cc: (0, cc))],
        out_specs=pl.BlockSpec((BM, 128), lambda i, cc: (cc * nblk + i, 0)),
        out_shape=jax.ShapeDtypeStruct((NC * N, 128), f32),
    )(X, W1)

    conv = _make_conv()
    x1 = conv(xw1, src, dst, norm, b1)

    xw2 = pl.pallas_call(
        _mm2_body,
        grid=(nblk, NC),
        in_specs=[pl.BlockSpec((BM, 128), lambda i, cc: (i, 0)),
                  pl.BlockSpec((BM, 128), lambda i, cc: (nblk + i, 0)),
                  pl.BlockSpec((H, 128), lambda i, cc: (0, cc))],
        out_specs=pl.BlockSpec((BM, 128), lambda i, cc: (cc * nblk + i, 0)),
        out_shape=jax.ShapeDtypeStruct((NC * N, 128), f32),
    )(x1, x1, W2)

    x2 = conv(xw2, src, dst, norm, b2)

    bb1 = b_ih1 + b_hh1
    Wt1 = jnp.concatenate([W_ih1[0:H], W_ih1[2 * H:4 * H]], axis=0).T
    bb1 = jnp.concatenate([bb1[0:H], bb1[2 * H:4 * H]]).reshape(1, 3 * H)
    bb2 = b_ih2 + b_hh2
    Wt2 = jnp.concatenate([W_ih2[0:H], W_ih2[2 * H:4 * H]], axis=0).T
    bb2 = jnp.concatenate([bb2[0:H], bb2[2 * H:4 * H]]).reshape(1, 3 * H)

    out = pl.pallas_call(
        _lstm_body,
        grid=(nblk,),
        in_specs=[pl.BlockSpec((BM, 128), lambda i: (i, 0)),
                  pl.BlockSpec((BM, 128), lambda i: (nblk + i, 0)),
                  pl.BlockSpec((BM, 128), lambda i: (i, 0)),
                  pl.BlockSpec((BM, 128), lambda i: (nblk + i, 0)),
                  pl.BlockSpec((BM, C), lambda i: (i, 0)),
                  pl.BlockSpec((2 * H, 3 * H), lambda i: (0, 0)),
                  pl.BlockSpec((1, 3 * H), lambda i: (0, 0)),
                  pl.BlockSpec((H, 3 * H), lambda i: (0, 0)),
                  pl.BlockSpec((1, 3 * H), lambda i: (0, 0)),
                  ],
        out_specs=pl.BlockSpec((BM, 2 * H + C), lambda i: (i, 0)),
        out_shape=jax.ShapeDtypeStruct((N, 2 * H + C), f32),
    )(x1, x1, x2, x2, X, Wt1, bb1, Wt2, bb2)
    return out


# speedup vs baseline: 11.7769x; 1.9445x over previous
import functools

import jax
import jax.numpy as jnp
from jax import lax
from jax.experimental import pallas as pl
from jax.experimental.pallas import tpu as pltpu
from jax.experimental.pallas import tpu_sc as plsc

N = 10000
C = 128
H = 256
K = 128
NT = 16
NC = 2
NPAD = NT * 640
BM = 1000
f32 = jnp.float32
i32 = jnp.int32

_mesh = plsc.VectorSubcoreMesh(core_axis_name="c", subcore_axis_name="s")


def _fast_rsqrt(x):
    bi = lax.bitcast_convert_type(x, i32)
    bi = 0x5F3759DF - lax.shift_right_arithmetic(bi, 1)
    y = lax.bitcast_convert_type(bi, f32)
    for _ in range(3):
        y = y * (1.5 - 0.5 * x * y * y)
    return y


def _prep_body(src_h, dst_h, w_h, norm_h, degS,
               idxA0, idxA1, idxB0, idxB1, valA0, valA1, degv, disv, normv,
               lsem0, lsem1):
    c = lax.axis_index("c")
    s = lax.axis_index("s")
    E2 = src_h.shape[0]
    npt = E2 // NT
    ncw = E2 // (NT * NC)
    idxA = (idxA0, idxA1)
    idxB = (idxB0, idxB1)
    valA = (valA0, valA1)
    lsem = (lsem0, lsem1)

    zero16 = jnp.zeros((16,), f32)
    for j in range(8):
        valA0[pl.ds(j * 16, 16)] = zero16
    for r in range(5):
        pltpu.sync_copy(valA0, degS.at[pl.ds(s * 640 + r * K, K)])
    plsc.subcore_barrier()

    nch = npt // K

    def fire2(ci, sl_):
        base = s * npt + ci * K
        pltpu.async_copy(dst_h.at[pl.ds(base, K)], idxA[sl_], lsem[sl_])
        pltpu.async_copy(w_h.at[pl.ds(base, K)], valA[sl_], lsem[sl_])

    def wait2(ci, sl_):
        base = s * npt + ci * K
        pltpu.make_async_copy(dst_h.at[pl.ds(base, K)], idxA[sl_], lsem[sl_]).wait()
        pltpu.make_async_copy(w_h.at[pl.ds(base, K)], valA[sl_], lsem[sl_]).wait()

    fire2(0, 0)

    def pair2(p, carry):
        for sl_ in (0, 1):
            ci = p * 2 + sl_
            wait2(ci, sl_)

            @pl.when(ci <= nch - 2)
            def _stage():
                fire2(ci + 1, 1 - sl_)

            pltpu.sync_copy(valA[sl_], degS.at[idxA[sl_]], add=True)
        return carry

    lax.fori_loop(0, nch // 2, pair2, 0)
    plsc.subcore_barrier()

    pltpu.sync_copy(degS.at[pl.ds(s * 640, 640)], degv)
    for j in range(40):
        sl = pl.ds(j * 16, 16)
        degv[sl] = _fast_rsqrt(degv[sl])
    pltpu.sync_copy(degv, degS.at[pl.ds(s * 640, 640)])
    plsc.subcore_barrier()

    pltpu.sync_copy(degS.at[pl.ds(0, N)], disv)
    wid = c * NT + s
    nchw = ncw // K

    def fire3(ci, sl_):
        base = wid * ncw + ci * K
        pltpu.async_copy(src_h.at[pl.ds(base, K)], idxA[sl_], lsem[sl_])
        pltpu.async_copy(dst_h.at[pl.ds(base, K)], idxB[sl_], lsem[sl_])
        pltpu.async_copy(w_h.at[pl.ds(base, K)], valA[sl_], lsem[sl_])

    def wait3(ci, sl_):
        base = wid * ncw + ci * K
        pltpu.make_async_copy(src_h.at[pl.ds(base, K)], idxA[sl_], lsem[sl_]).wait()
        pltpu.make_async_copy(dst_h.at[pl.ds(base, K)], idxB[sl_], lsem[sl_]).wait()
        pltpu.make_async_copy(w_h.at[pl.ds(base, K)], valA[sl_], lsem[sl_]).wait()

    def norm_chunk(ci, sl_):
        wait3(ci, sl_)

        @pl.when(ci <= nchw - 2)
        def _stage():
            fire3(ci + 1, 1 - sl_)

        for k in range(8):
            sl = pl.ds(k * 16, 16)
            nm = plsc.load_gather(disv, [idxA[sl_][sl]]) * valA[sl_][sl] \
                * plsc.load_gather(disv, [idxB[sl_][sl]])
            normv[sl] = nm
        base = wid * ncw + ci * K
        pltpu.sync_copy(normv, norm_h.at[pl.ds(base, K)])

    fire3(0, 0)

    def pair3(p, carry):
        for sl_ in (0, 1):
            norm_chunk(p * 2 + sl_, sl_)
        return carry

    lax.fori_loop(0, (nchw - 1) // 2, pair3, 0)
    norm_chunk(nchw - 1, (nchw - 1) % 2)


def _conv_body(xw_h, src_h, dst_h, norm_h, b_h, out_h, accS,
               idx0, idx1, dst0, dst1, nrm0, nrm1, rows0, rows1, biasv,
               lds0, lds1, dss0, dss1, gts0, gts1, scs0, scs1):
    c = lax.axis_index("c")
    s = lax.axis_index("s")
    E2 = src_h.shape[0]
    npt = E2 // NT
    nch = npt // K
    coff = c * N
    idxs = (idx0, idx1)
    dsts = (dst0, dst1)
    nrms = (nrm0, nrm1)
    rows = (rows0, rows1)
    ldss = (lds0, lds1)
    dsss = (dss0, dss1)
    gtss = (gts0, gts1)
    scss = (scs0, scs1)

    def ebase(ci):
        return s * npt + ci * K

    def fire_sn(ci, sl_):
        pltpu.async_copy(src_h.at[pl.ds(ebase(ci), K)], idxs[sl_], ldss[sl_])
        pltpu.async_copy(norm_h.at[pl.ds(ebase(ci), K)], nrms[sl_], ldss[sl_])

    def wait_sn(ci, sl_):
        pltpu.make_async_copy(src_h.at[pl.ds(ebase(ci), K)], idxs[sl_], ldss[sl_]).wait()
        pltpu.make_async_copy(norm_h.at[pl.ds(ebase(ci), K)], nrms[sl_], ldss[sl_]).wait()

    def fire_dst(ci, sl_):
        pltpu.async_copy(dst_h.at[pl.ds(ebase(ci), K)], dsts[sl_], dsss[sl_])

    def wait_dst(ci, sl_):
        pltpu.make_async_copy(dst_h.at[pl.ds(ebase(ci), K)], dsts[sl_], dsss[sl_]).wait()

    def adjust(sl_):
        for j in range(8):
            sl = pl.ds(j * 16, 16)
            idxs[sl_][sl] = idxs[sl_][sl] + coff

    def fire_gather(sl_):
        pltpu.async_copy(xw_h.at[idxs[sl_]], rows[sl_], gtss[sl_])

    def wait_gather(sl_):
        pltpu.make_async_copy(xw_h.at[idxs[sl_]], rows[sl_], gtss[sl_]).wait()

    def fire_scatter(sl_):
        pltpu.async_copy(rows[sl_], accS.at[dsts[sl_]], scss[sl_], add=True)

    def wait_scatter(sl_):
        pltpu.make_async_copy(rows[sl_], accS.at[dsts[sl_]], scss[sl_]).wait()

    def scale(sl_):
        def scale16(k16, carry):
            for l in range(16):
                k = k16 * 16 + l
                nb = plsc.load_gather(nrms[sl_], [jnp.zeros((16,), i32) + k])
                for j in range(8):
                    slc = pl.ds(j * 16, 16)
                    rows[sl_][k, slc] = rows[sl_][k, slc] * nb
            return carry

        lax.fori_loop(0, K // 16, scale16, 0)

    zero16 = jnp.zeros((16,), f32)

    def zrow(r, carry):
        for j in range(8):
            rows0[r, pl.ds(j * 16, 16)] = zero16
        return carry

    lax.fori_loop(0, K, zrow, 0)
    for r in range(5):
        pltpu.sync_copy(rows0, accS.at[pl.ds(s * 640 + r * K, K)])
    plsc.subcore_barrier()
    pltpu.sync_copy(b_h.at[pl.ds(c * 128, 128)], biasv)

    fire_sn(0, 0)
    fire_dst(0, 0)
    fire_sn(1, 1)
    wait_sn(0, 0)
    adjust(0)
    fire_gather(0)

    def pair(p, carry):
        for sl_ in (0, 1):
            so = 1 - sl_
            ci = p * 2 + sl_

            @pl.when(ci >= 1)
            def _drain():
                wait_scatter(so)

            @pl.when(ci <= nch - 2)
            def _stage():
                wait_sn(ci + 1, so)
                adjust(so)
                fire_dst(ci + 1, so)
                fire_gather(so)

            wait_gather(sl_)
            scale(sl_)
            wait_dst(ci, sl_)
            fire_scatter(sl_)

            @pl.when(ci <= nch - 3)
            def _prefetch():
                fire_sn(ci + 2, sl_)
        return carry

    lax.fori_loop(0, nch // 2, pair, 0)
    wait_scatter(1)
    plsc.subcore_barrier()

    bvecs = [biasv[pl.ds(j * 16, 16)] for j in range(8)]

    def ep(r, carry):
        for j in range(8):
            sl = pl.ds(j * 16, 16)
            rows0[r, sl] = jnp.maximum(rows0[r, sl] + bvecs[j], 0.0)
        return carry

    def do_piece(r0, ln):
        pltpu.sync_copy(accS.at[pl.ds(r0, ln)], rows0.at[pl.ds(0, ln)])
        lax.fori_loop(0, ln, ep, 0)
        pltpu.sync_copy(rows0.at[pl.ds(0, ln)], out_h.at[pl.ds(c * N + r0, ln)])

    base = s * 624
    for off, ln in ((0, K), (K, K), (2 * K, K), (3 * K, K), (4 * K, 112)):
        do_piece(base + off, ln)

    @pl.when(s == NT - 1)
    def _tail():
        do_piece(9984, 16)


_sc_params = pltpu.CompilerParams(needs_layout_passes=False)


def _make_prep(E2):
    return functools.partial(
        pl.kernel,
        out_type=jax.ShapeDtypeStruct((E2,), f32),
        mesh=_mesh,
        compiler_params=_sc_params,
        scratch_types=[
            pltpu.VMEM_SHARED((NPAD,), f32),
            pltpu.VMEM((K,), i32),
            pltpu.VMEM((K,), i32),
            pltpu.VMEM((K,), i32),
            pltpu.VMEM((K,), i32),
            pltpu.VMEM((K,), f32),
            pltpu.VMEM((K,), f32),
            pltpu.VMEM((640,), f32),
            pltpu.VMEM((N,), f32),
            pltpu.VMEM((K,), f32),
            pltpu.SemaphoreType.DMA,
            pltpu.SemaphoreType.DMA,
        ],
    )(_prep_body)


def _make_conv():
    return functools.partial(
        pl.kernel,
        out_type=jax.ShapeDtypeStruct((NC * N, 128), f32),
        mesh=_mesh,
        compiler_params=_sc_params,
        scratch_types=[
            pltpu.VMEM_SHARED((NPAD, 128), f32),
            pltpu.VMEM((K,), i32),
            pltpu.VMEM((K,), i32),
            pltpu.VMEM((K,), i32),
            pltpu.VMEM((K,), i32),
            pltpu.VMEM((K,), f32),
            pltpu.VMEM((K,), f32),
            pltpu.VMEM((K, 128), f32),
            pltpu.VMEM((K, 128), f32),
            pltpu.VMEM((128,), f32),
            pltpu.SemaphoreType.DMA,
            pltpu.SemaphoreType.DMA,
            pltpu.SemaphoreType.DMA,
            pltpu.SemaphoreType.DMA,
            pltpu.SemaphoreType.DMA,
            pltpu.SemaphoreType.DMA,
            pltpu.SemaphoreType.DMA,
            pltpu.SemaphoreType.DMA,
        ],
    )(_conv_body)


def _mm1_body(x_ref, w_ref, o_ref):
    o_ref[...] = jnp.dot(x_ref[...], w_ref[...], preferred_element_type=f32)


def _mm2_body(xa_ref, xb_ref, w_ref, o_ref):
    o_ref[...] = (jnp.dot(xa_ref[...], w_ref[0:128], preferred_element_type=f32)
                  + jnp.dot(xb_ref[...], w_ref[128:256], preferred_element_type=f32))


def _lstm_body(x1a, x1b, x2a, x2b, x_ref, wt1, bb1, wt2, bb2, o_ref):
    g1 = (jnp.dot(x1a[...], wt1[0:128], preferred_element_type=f32)
          + jnp.dot(x1b[...], wt1[128:256], preferred_element_type=f32)
          + jnp.dot(x2a[...], wt1[256:384], preferred_element_type=f32)
          + jnp.dot(x2b[...], wt1[384:512], preferred_element_type=f32)) + bb1[...]
    gi = g1[:, 0:H]
    gg = g1[:, H:2 * H]
    go = g1[:, 2 * H:3 * H]
    cell = jax.nn.sigmoid(gi) * jnp.tanh(gg)
    h1 = jax.nn.sigmoid(go) * jnp.tanh(cell)
    g2 = jnp.dot(h1, wt2[...], preferred_element_type=f32) + bb2[...]
    gi2 = g2[:, 0:H]
    gg2 = g2[:, H:2 * H]
    go2 = g2[:, 2 * H:3 * H]
    cell2 = jax.nn.sigmoid(gi2) * jnp.tanh(gg2)
    h2 = jax.nn.sigmoid(go2) * jnp.tanh(cell2)
    o_ref[:, 0:H] = h1
    o_ref[:, H:2 * H] = h2
    o_ref[:, 2 * H:2 * H + C] = x_ref[...]


def kernel(X, edge_index, edge_weight, W1, b1, W2, b2,
           W_ih1, W_hh1, b_ih1, b_hh1, W_ih2, W_hh2, b_ih2, b_hh2):
    E = edge_weight.shape[0]
    loop = jnp.arange(N, dtype=edge_index.dtype)
    src = jnp.concatenate([edge_index[0], loop])
    dst = jnp.concatenate([edge_index[1], loop])
    w = jnp.concatenate([edge_weight, jnp.ones((N,), f32)])
    E2 = ((E + N + NT * NC * K - 1) // (NT * NC * K)) * (NT * NC * K)
    pad = E2 - (E + N)
    if pad:
        src = jnp.concatenate([src, jnp.zeros((pad,), src.dtype)])
        dst = jnp.concatenate([dst, jnp.zeros((pad,), dst.dtype)])
        w = jnp.concatenate([w, jnp.zeros((pad,), f32)])

    norm = _make_prep(E2)(src, dst, w)

    nblk = N // BM
    xw1 = pl.pallas_call(
        _mm1_body,
        grid=(nblk, NC),
        in_specs=[pl.BlockSpec((BM, C), lambda i, cc: (i, 0)),
                  pl.BlockSpec((C, 128), lambda i, cc: (0, cc))],
        out_specs=pl.BlockSpec((BM, 128), lambda i, cc: (cc * nblk + i, 0)),
        out_shape=jax.ShapeDtypeStruct((NC * N, 128), f32),
    )(X, W1)

    conv = _make_conv()
    x1 = conv(xw1, src, dst, norm, b1)

    xw2 = pl.pallas_call(
        _mm2_body,
        grid=(nblk, NC),
        in_specs=[pl.BlockSpec((BM, 128), lambda i, cc: (i, 0)),
                  pl.BlockSpec((BM, 128), lambda i, cc: (nblk + i, 0)),
                  pl.BlockSpec((H, 128), lambda i, cc: (0, cc))],
        out_specs=pl.BlockSpec((BM, 128), lambda i, cc: (cc * nblk + i, 0)),
        out_shape=jax.ShapeDtypeStruct((NC * N, 128), f32),
    )(x1, x1, W2)

    x2 = conv(xw2, src, dst, norm, b2)

    bb1 = b_ih1 + b_hh1
    Wt1 = jnp.concatenate([W_ih1[0:H], W_ih1[2 * H:4 * H]], axis=0).T
    bb1 = jnp.concatenate([bb1[0:H], bb1[2 * H:4 * H]]).reshape(1, 3 * H)
    bb2 = b_ih2 + b_hh2
    Wt2 = jnp.concatenate([W_ih2[0:H], W_ih2[2 * H:4 * H]], axis=0).T
    bb2 = jnp.concatenate([bb2[0:H], bb2[2 * H:4 * H]]).reshape(1, 3 * H)

    out = pl.pallas_call(
        _lstm_body,
        grid=(nblk,),
        in_specs=[pl.BlockSpec((BM, 128), lambda i: (i, 0)),
                  pl.BlockSpec((BM, 128), lambda i: (nblk + i, 0)),
                  pl.BlockSpec((BM, 128), lambda i: (i, 0)),
                  pl.BlockSpec((BM, 128), lambda i: (nblk + i, 0)),
                  pl.BlockSpec((BM, C), lambda i: (i, 0)),
                  pl.BlockSpec((2 * H, 3 * H), lambda i: (0, 0)),
                  pl.BlockSpec((1, 3 * H), lambda i: (0, 0)),
                  pl.BlockSpec((H, 3 * H), lambda i: (0, 0)),
                  pl.BlockSpec((1, 3 * H), lambda i: (0, 0)),
                  ],
        out_specs=pl.BlockSpec((BM, 2 * H + C), lambda i: (i, 0)),
        out_shape=jax.ShapeDtypeStruct((N, 2 * H + C), f32),
    )(x1, x1, x2, x2, X, Wt1, bb1, Wt2, bb2)
    return out


# speedup vs baseline: 13.3960x; 1.1375x over previous
import functools

import jax
import jax.numpy as jnp
from jax import lax
from jax.experimental import pallas as pl
from jax.experimental.pallas import tpu as pltpu
from jax.experimental.pallas import tpu_sc as plsc

N = 10000
C = 128
H = 256
K = 128
NT = 16
NC = 2
NPAD = NT * 640
BM = 1000
f32 = jnp.float32
i32 = jnp.int32

_mesh = plsc.VectorSubcoreMesh(core_axis_name="c", subcore_axis_name="s")


def _fast_rsqrt(x):
    bi = lax.bitcast_convert_type(x, i32)
    bi = 0x5F3759DF - lax.shift_right_arithmetic(bi, 1)
    y = lax.bitcast_convert_type(bi, f32)
    for _ in range(3):
        y = y * (1.5 - 0.5 * x * y * y)
    return y


def _prep_body(src_h, dst_h, w_h, norm_h, degS,
               idxA0, idxA1, idxB0, idxB1, valA0, valA1, degv, disv, normv,
               lsem0, lsem1, ssem0, ssem1):
    c = lax.axis_index("c")
    s = lax.axis_index("s")
    E2 = src_h.shape[0]
    npt = E2 // NT
    ncw = E2 // (NT * NC)
    idxA = (idxA0, idxA1)
    idxB = (idxB0, idxB1)
    valA = (valA0, valA1)
    lsem = (lsem0, lsem1)
    ssem = (ssem0, ssem1)

    zero16 = jnp.zeros((16,), f32)
    for j in range(8):
        valA0[pl.ds(j * 16, 16)] = zero16
    for r in range(5):
        pltpu.sync_copy(valA0, degS.at[pl.ds(s * 640 + r * K, K)])
    plsc.subcore_barrier()

    nch = npt // K

    def fire2(ci, sl_):
        base = s * npt + ci * K
        pltpu.async_copy(dst_h.at[pl.ds(base, K)], idxA[sl_], lsem[sl_])
        pltpu.async_copy(w_h.at[pl.ds(base, K)], valA[sl_], lsem[sl_])

    def wait2(ci, sl_):
        base = s * npt + ci * K
        pltpu.make_async_copy(dst_h.at[pl.ds(base, K)], idxA[sl_], lsem[sl_]).wait()
        pltpu.make_async_copy(w_h.at[pl.ds(base, K)], valA[sl_], lsem[sl_]).wait()

    def fire_sc2(sl_):
        pltpu.async_copy(valA[sl_], degS.at[idxA[sl_]], ssem[sl_], add=True)

    def wait_sc2(sl_):
        pltpu.make_async_copy(valA[sl_], degS.at[idxA[sl_]], ssem[sl_]).wait()

    fire2(0, 0)

    def pair2(p, carry):
        for sl_ in (0, 1):
            ci = p * 2 + sl_
            so = 1 - sl_
            wait2(ci, sl_)

            @pl.when(ci >= 1)
            def _drain():
                wait_sc2(so)

            @pl.when(ci <= nch - 2)
            def _stage():
                fire2(ci + 1, so)

            fire_sc2(sl_)
        return carry

    lax.fori_loop(0, nch // 2, pair2, 0)
    wait_sc2(1)
    plsc.subcore_barrier()

    pltpu.sync_copy(degS.at[pl.ds(s * 640, 640)], degv)
    for j in range(40):
        sl = pl.ds(j * 16, 16)
        degv[sl] = _fast_rsqrt(degv[sl])
    pltpu.sync_copy(degv, degS.at[pl.ds(s * 640, 640)])
    plsc.subcore_barrier()

    pltpu.sync_copy(degS.at[pl.ds(0, N)], disv)
    wid = c * NT + s
    nchw = ncw // K

    def fire3(ci, sl_):
        base = wid * ncw + ci * K
        pltpu.async_copy(src_h.at[pl.ds(base, K)], idxA[sl_], lsem[sl_])
        pltpu.async_copy(dst_h.at[pl.ds(base, K)], idxB[sl_], lsem[sl_])
        pltpu.async_copy(w_h.at[pl.ds(base, K)], valA[sl_], lsem[sl_])

    def wait3(ci, sl_):
        base = wid * ncw + ci * K
        pltpu.make_async_copy(src_h.at[pl.ds(base, K)], idxA[sl_], lsem[sl_]).wait()
        pltpu.make_async_copy(dst_h.at[pl.ds(base, K)], idxB[sl_], lsem[sl_]).wait()
        pltpu.make_async_copy(w_h.at[pl.ds(base, K)], valA[sl_], lsem[sl_]).wait()

    def norm_chunk(ci, sl_):
        wait3(ci, sl_)

        @pl.when(ci <= nchw - 2)
        def _stage():
            fire3(ci + 1, 1 - sl_)

        for k in range(8):
            sl = pl.ds(k * 16, 16)
            nm = plsc.load_gather(disv, [idxA[sl_][sl]]) * valA[sl_][sl] \
                * plsc.load_gather(disv, [idxB[sl_][sl]])
            normv[sl] = nm
        base = wid * ncw + ci * K
        pltpu.sync_copy(normv, norm_h.at[pl.ds(base, K)])

    fire3(0, 0)

    def pair3(p, carry):
        for sl_ in (0, 1):
            norm_chunk(p * 2 + sl_, sl_)
        return carry

    lax.fori_loop(0, (nchw - 1) // 2, pair3, 0)
    norm_chunk(nchw - 1, (nchw - 1) % 2)


def _conv_body(xw_h, src_h, dst_h, norm_h, b_h, out_h, accS,
               idx0, idx1, dst0, dst1, nrm0, nrm1, rows0, rows1, biasv,
               lds0, lds1, dss0, dss1, gts0, gts1, scs0, scs1):
    c = lax.axis_index("c")
    s = lax.axis_index("s")
    E2 = src_h.shape[0]
    npt = E2 // NT
    nch = npt // K
    coff = c * N
    idxs = (idx0, idx1)
    dsts = (dst0, dst1)
    nrms = (nrm0, nrm1)
    rows = (rows0, rows1)
    ldss = (lds0, lds1)
    dsss = (dss0, dss1)
    gtss = (gts0, gts1)
    scss = (scs0, scs1)

    def ebase(ci):
        return s * npt + ci * K

    def fire_sn(ci, sl_):
        pltpu.async_copy(src_h.at[pl.ds(ebase(ci), K)], idxs[sl_], ldss[sl_])
        pltpu.async_copy(norm_h.at[pl.ds(ebase(ci), K)], nrms[sl_], ldss[sl_])

    def wait_sn(ci, sl_):
        pltpu.make_async_copy(src_h.at[pl.ds(ebase(ci), K)], idxs[sl_], ldss[sl_]).wait()
        pltpu.make_async_copy(norm_h.at[pl.ds(ebase(ci), K)], nrms[sl_], ldss[sl_]).wait()

    def fire_dst(ci, sl_):
        pltpu.async_copy(dst_h.at[pl.ds(ebase(ci), K)], dsts[sl_], dsss[sl_])

    def wait_dst(ci, sl_):
        pltpu.make_async_copy(dst_h.at[pl.ds(ebase(ci), K)], dsts[sl_], dsss[sl_]).wait()

    def adjust(sl_):
        for j in range(8):
            sl = pl.ds(j * 16, 16)
            idxs[sl_][sl] = idxs[sl_][sl] + coff

    def fire_gather(sl_):
        pltpu.async_copy(xw_h.at[idxs[sl_]], rows[sl_], gtss[sl_])

    def wait_gather(sl_):
        pltpu.make_async_copy(xw_h.at[idxs[sl_]], rows[sl_], gtss[sl_]).wait()

    def fire_scatter(sl_):
        pltpu.async_copy(rows[sl_], accS.at[dsts[sl_]], scss[sl_], add=True)

    def wait_scatter(sl_):
        pltpu.make_async_copy(rows[sl_], accS.at[dsts[sl_]], scss[sl_]).wait()

    def scale(sl_):
        @plsc.parallel_loop(0, K, unroll=8)
        def _body(k):
            nb = plsc.load_gather(nrms[sl_], [jnp.zeros((16,), i32) + k])
            for j in range(8):
                slc = pl.ds(j * 16, 16)
                rows[sl_][k, slc] = rows[sl_][k, slc] * nb

    zero16 = jnp.zeros((16,), f32)

    @plsc.parallel_loop(0, K, unroll=8)
    def _zrow(r):
        for j in range(8):
            rows0[r, pl.ds(j * 16, 16)] = zero16
    for r in range(5):
        pltpu.sync_copy(rows0, accS.at[pl.ds(s * 640 + r * K, K)])
    plsc.subcore_barrier()
    pltpu.sync_copy(b_h.at[pl.ds(c * 128, 128)], biasv)

    fire_sn(0, 0)
    fire_dst(0, 0)
    fire_sn(1, 1)
    wait_sn(0, 0)
    adjust(0)
    fire_gather(0)

    def pair(p, carry):
        for sl_ in (0, 1):
            so = 1 - sl_
            ci = p * 2 + sl_

            @pl.when(ci >= 1)
            def _drain():
                wait_scatter(so)

            @pl.when(ci <= nch - 2)
            def _stage():
                wait_sn(ci + 1, so)
                adjust(so)
                fire_dst(ci + 1, so)
                fire_gather(so)

            wait_gather(sl_)
            scale(sl_)
            wait_dst(ci, sl_)
            fire_scatter(sl_)

            @pl.when(ci <= nch - 3)
            def _prefetch():
                fire_sn(ci + 2, sl_)
        return carry

    lax.fori_loop(0, nch // 2, pair, 0)
    wait_scatter(1)
    plsc.subcore_barrier()

    bvecs = [biasv[pl.ds(j * 16, 16)] for j in range(8)]

    def do_piece(r0, ln):
        pltpu.sync_copy(accS.at[pl.ds(r0, ln)], rows0.at[pl.ds(0, ln)])

        @plsc.parallel_loop(0, ln, unroll=4)
        def _ep(r):
            for j in range(8):
                sl = pl.ds(j * 16, 16)
                rows0[r, sl] = jnp.maximum(rows0[r, sl] + bvecs[j], 0.0)

        pltpu.sync_copy(rows0.at[pl.ds(0, ln)], out_h.at[pl.ds(c * N + r0, ln)])

    base = s * 624
    for off, ln in ((0, K), (K, K), (2 * K, K), (3 * K, K), (4 * K, 112)):
        do_piece(base + off, ln)

    @pl.when(s == NT - 1)
    def _tail():
        do_piece(9984, 16)


_sc_params = pltpu.CompilerParams(needs_layout_passes=False)


def _make_prep(E2):
    return functools.partial(
        pl.kernel,
        out_type=jax.ShapeDtypeStruct((E2,), f32),
        mesh=_mesh,
        compiler_params=_sc_params,
        scratch_types=[
            pltpu.VMEM_SHARED((NPAD,), f32),
            pltpu.VMEM((K,), i32),
            pltpu.VMEM((K,), i32),
            pltpu.VMEM((K,), i32),
            pltpu.VMEM((K,), i32),
            pltpu.VMEM((K,), f32),
            pltpu.VMEM((K,), f32),
            pltpu.VMEM((640,), f32),
            pltpu.VMEM((N,), f32),
            pltpu.VMEM((K,), f32),
            pltpu.SemaphoreType.DMA,
            pltpu.SemaphoreType.DMA,
            pltpu.SemaphoreType.DMA,
            pltpu.SemaphoreType.DMA,
        ],
    )(_prep_body)


def _make_conv():
    return functools.partial(
        pl.kernel,
        out_type=jax.ShapeDtypeStruct((NC * N, 128), f32),
        mesh=_mesh,
        compiler_params=_sc_params,
        scratch_types=[
            pltpu.VMEM_SHARED((NPAD, 128), f32),
            pltpu.VMEM((K,), i32),
            pltpu.VMEM((K,), i32),
            pltpu.VMEM((K,), i32),
            pltpu.VMEM((K,), i32),
            pltpu.VMEM((K,), f32),
            pltpu.VMEM((K,), f32),
            pltpu.VMEM((K, 128), f32),
            pltpu.VMEM((K, 128), f32),
            pltpu.VMEM((128,), f32),
            pltpu.SemaphoreType.DMA,
            pltpu.SemaphoreType.DMA,
            pltpu.SemaphoreType.DMA,
            pltpu.SemaphoreType.DMA,
            pltpu.SemaphoreType.DMA,
            pltpu.SemaphoreType.DMA,
            pltpu.SemaphoreType.DMA,
            pltpu.SemaphoreType.DMA,
        ],
    )(_conv_body)


def _mm1_body(x_ref, w_ref, o_ref):
    o_ref[...] = jnp.dot(x_ref[...], w_ref[...], preferred_element_type=f32)


def _mm2_body(xa_ref, xb_ref, w_ref, o_ref):
    o_ref[...] = (jnp.dot(xa_ref[...], w_ref[0:128], preferred_element_type=f32)
                  + jnp.dot(xb_ref[...], w_ref[128:256], preferred_element_type=f32))


def _lstm_body(x1a, x1b, x2a, x2b, x_ref, wt1, bb1, wt2, bb2, o_ref):
    g1 = (jnp.dot(x1a[...], wt1[0:128], preferred_element_type=f32)
          + jnp.dot(x1b[...], wt1[128:256], preferred_element_type=f32)
          + jnp.dot(x2a[...], wt1[256:384], preferred_element_type=f32)
          + jnp.dot(x2b[...], wt1[384:512], preferred_element_type=f32)) + bb1[...]
    gi = g1[:, 0:H]
    gg = g1[:, H:2 * H]
    go = g1[:, 2 * H:3 * H]
    cell = jax.nn.sigmoid(gi) * jnp.tanh(gg)
    h1 = jax.nn.sigmoid(go) * jnp.tanh(cell)
    g2 = jnp.dot(h1, wt2[...], preferred_element_type=f32) + bb2[...]
    gi2 = g2[:, 0:H]
    gg2 = g2[:, H:2 * H]
    go2 = g2[:, 2 * H:3 * H]
    cell2 = jax.nn.sigmoid(gi2) * jnp.tanh(gg2)
    h2 = jax.nn.sigmoid(go2) * jnp.tanh(cell2)
    o_ref[:, 0:H] = h1
    o_ref[:, H:2 * H] = h2
    o_ref[:, 2 * H:2 * H + C] = x_ref[...]


def kernel(X, edge_index, edge_weight, W1, b1, W2, b2,
           W_ih1, W_hh1, b_ih1, b_hh1, W_ih2, W_hh2, b_ih2, b_hh2):
    E = edge_weight.shape[0]
    loop = jnp.arange(N, dtype=edge_index.dtype)
    src = jnp.concatenate([edge_index[0], loop])
    dst = jnp.concatenate([edge_index[1], loop])
    w = jnp.concatenate([edge_weight, jnp.ones((N,), f32)])
    E2 = ((E + N + NT * NC * K - 1) // (NT * NC * K)) * (NT * NC * K)
    pad = E2 - (E + N)
    if pad:
        src = jnp.concatenate([src, jnp.zeros((pad,), src.dtype)])
        dst = jnp.concatenate([dst, jnp.zeros((pad,), dst.dtype)])
        w = jnp.concatenate([w, jnp.zeros((pad,), f32)])

    norm = _make_prep(E2)(src, dst, w)

    nblk = N // BM
    xw1 = pl.pallas_call(
        _mm1_body,
        grid=(nblk, NC),
        in_specs=[pl.BlockSpec((BM, C), lambda i, cc: (i, 0)),
                  pl.BlockSpec((C, 128), lambda i, cc: (0, cc))],
        out_specs=pl.BlockSpec((BM, 128), lambda i, cc: (cc * nblk + i, 0)),
        out_shape=jax.ShapeDtypeStruct((NC * N, 128), f32),
    )(X, W1)

    conv = _make_conv()
    x1 = conv(xw1, src, dst, norm, b1)

    xw2 = pl.pallas_call(
        _mm2_body,
        grid=(nblk, NC),
        in_specs=[pl.BlockSpec((BM, 128), lambda i, cc: (i, 0)),
                  pl.BlockSpec((BM, 128), lambda i, cc: (nblk + i, 0)),
                  pl.BlockSpec((H, 128), lambda i, cc: (0, cc))],
        out_specs=pl.BlockSpec((BM, 128), lambda i, cc: (cc * nblk + i, 0)),
        out_shape=jax.ShapeDtypeStruct((NC * N, 128), f32),
    )(x1, x1, W2)

    x2 = conv(xw2, src, dst, norm, b2)

    bb1 = b_ih1 + b_hh1
    Wt1 = jnp.concatenate([W_ih1[0:H], W_ih1[2 * H:4 * H]], axis=0).T
    bb1 = jnp.concatenate([bb1[0:H], bb1[2 * H:4 * H]]).reshape(1, 3 * H)
    bb2 = b_ih2 + b_hh2
    Wt2 = jnp.concatenate([W_ih2[0:H], W_ih2[2 * H:4 * H]], axis=0).T
    bb2 = jnp.concatenate([bb2[0:H], bb2[2 * H:4 * H]]).reshape(1, 3 * H)

    out = pl.pallas_call(
        _lstm_body,
        grid=(nblk,),
        in_specs=[pl.BlockSpec((BM, 128), lambda i: (i, 0)),
                  pl.BlockSpec((BM, 128), lambda i: (nblk + i, 0)),
                  pl.BlockSpec((BM, 128), lambda i: (i, 0)),
                  pl.BlockSpec((BM, 128), lambda i: (nblk + i, 0)),
                  pl.BlockSpec((BM, C), lambda i: (i, 0)),
                  pl.BlockSpec((2 * H, 3 * H), lambda i: (0, 0)),
                  pl.BlockSpec((1, 3 * H), lambda i: (0, 0)),
                  pl.BlockSpec((H, 3 * H), lambda i: (0, 0)),
                  pl.BlockSpec((1, 3 * H), lambda i: (0, 0)),
                  ],
        out_specs=pl.BlockSpec((BM, 2 * H + C), lambda i: (i, 0)),
        out_shape=jax.ShapeDtypeStruct((N, 2 * H + C), f32),
    )(x1, x1, x2, x2, X, Wt1, bb1, Wt2, bb2)
    return out


# speedup vs baseline: 13.6476x; 1.0188x over previous
import functools

import jax
import jax.numpy as jnp
from jax import lax
from jax.experimental import pallas as pl
from jax.experimental.pallas import tpu as pltpu
from jax.experimental.pallas import tpu_sc as plsc

N = 10000
C = 128
H = 256
K = 128
NT = 16
NC = 2
NPAD = 10016
BM = 1000
f32 = jnp.float32
i32 = jnp.int32

_mesh = plsc.VectorSubcoreMesh(core_axis_name="c", subcore_axis_name="s")


def _fast_rsqrt(x):
    bi = lax.bitcast_convert_type(x, i32)
    bi = 0x5F3759DF - lax.shift_right_arithmetic(bi, 1)
    y = lax.bitcast_convert_type(bi, f32)
    for _ in range(3):
        y = y * (1.5 - 0.5 * x * y * y)
    return y


def _prep_body(src_h, dst_h, w_h, norm_h, degS,
               idxA0, idxA1, idxB0, idxB1, valA0, valA1, degv, disv, normv,
               lsem0, lsem1, ssem0, ssem1):
    c = lax.axis_index("c")
    s = lax.axis_index("s")
    E2 = src_h.shape[0]
    npt = E2 // NT
    ncw = E2 // (NT * NC)
    idxA = (idxA0, idxA1)
    idxB = (idxB0, idxB1)
    valA = (valA0, valA1)
    lsem = (lsem0, lsem1)
    ssem = (ssem0, ssem1)

    zero16 = jnp.zeros((16,), f32)
    for j in range(8):
        valA0[pl.ds(j * 16, 16)] = zero16
    for r in range(5):
        pltpu.sync_copy(valA0, degS.at[pl.ds(s * 640 + r * K, K)])
    plsc.subcore_barrier()

    nch = npt // K

    def fire2(ci, sl_):
        base = s * npt + ci * K
        pltpu.async_copy(dst_h.at[pl.ds(base, K)], idxA[sl_], lsem[sl_])
        pltpu.async_copy(w_h.at[pl.ds(base, K)], valA[sl_], lsem[sl_])

    def wait2(ci, sl_):
        base = s * npt + ci * K
        pltpu.make_async_copy(dst_h.at[pl.ds(base, K)], idxA[sl_], lsem[sl_]).wait()
        pltpu.make_async_copy(w_h.at[pl.ds(base, K)], valA[sl_], lsem[sl_]).wait()

    def fire_sc2(sl_):
        pltpu.async_copy(valA[sl_], degS.at[idxA[sl_]], ssem[sl_], add=True)

    def wait_sc2(sl_):
        pltpu.make_async_copy(valA[sl_], degS.at[idxA[sl_]], ssem[sl_]).wait()

    fire2(0, 0)

    def pair2(p, carry):
        for sl_ in (0, 1):
            ci = p * 2 + sl_
            so = 1 - sl_
            wait2(ci, sl_)

            @pl.when(ci >= 1)
            def _drain():
                wait_sc2(so)

            @pl.when(ci <= nch - 2)
            def _stage():
                fire2(ci + 1, so)

            fire_sc2(sl_)
        return carry

    lax.fori_loop(0, nch // 2, pair2, 0)
    wait_sc2(1)
    plsc.subcore_barrier()

    pltpu.sync_copy(degS.at[pl.ds(s * 640, 640)], degv)
    for j in range(40):
        sl = pl.ds(j * 16, 16)
        degv[sl] = _fast_rsqrt(degv[sl])
    pltpu.sync_copy(degv, degS.at[pl.ds(s * 640, 640)])
    plsc.subcore_barrier()

    pltpu.sync_copy(degS.at[pl.ds(0, N)], disv)
    wid = c * NT + s
    nchw = ncw // K

    def fire3(ci, sl_):
        base = wid * ncw + ci * K
        pltpu.async_copy(src_h.at[pl.ds(base, K)], idxA[sl_], lsem[sl_])
        pltpu.async_copy(dst_h.at[pl.ds(base, K)], idxB[sl_], lsem[sl_])
        pltpu.async_copy(w_h.at[pl.ds(base, K)], valA[sl_], lsem[sl_])

    def wait3(ci, sl_):
        base = wid * ncw + ci * K
        pltpu.make_async_copy(src_h.at[pl.ds(base, K)], idxA[sl_], lsem[sl_]).wait()
        pltpu.make_async_copy(dst_h.at[pl.ds(base, K)], idxB[sl_], lsem[sl_]).wait()
        pltpu.make_async_copy(w_h.at[pl.ds(base, K)], valA[sl_], lsem[sl_]).wait()

    def norm_chunk(ci, sl_):
        wait3(ci, sl_)

        @pl.when(ci <= nchw - 2)
        def _stage():
            fire3(ci + 1, 1 - sl_)

        for k in range(8):
            sl = pl.ds(k * 16, 16)
            nm = plsc.load_gather(disv, [idxA[sl_][sl]]) * valA[sl_][sl] \
                * plsc.load_gather(disv, [idxB[sl_][sl]])
            normv[sl] = nm
        base = wid * ncw + ci * K
        pltpu.sync_copy(normv, norm_h.at[pl.ds(base, K)])

    fire3(0, 0)

    def pair3(p, carry):
        for sl_ in (0, 1):
            norm_chunk(p * 2 + sl_, sl_)
        return carry

    lax.fori_loop(0, nchw // 2, pair3, 0)
    if nchw % 2 == 1:
        norm_chunk(nchw - 1, (nchw - 1) % 2)


def _conv_body(xw_h, src_h, dst_h, norm_h, b_h, out_h, accS,
               idx0, idx1, idx2, dst0, dst1, dst2,
               nrm0, nrm1, nrm2, rows0, rows1, rows2, biasv,
               lds0, lds1, lds2, dss0, dss1, dss2,
               gts0, gts1, gts2, scs0, scs1, scs2):
    c = lax.axis_index("c")
    s = lax.axis_index("s")
    E2 = src_h.shape[0]
    npt = E2 // NT
    nch = npt // K
    coff = c * N
    idxs = (idx0, idx1, idx2)
    dsts = (dst0, dst1, dst2)
    nrms = (nrm0, nrm1, nrm2)
    rows = (rows0, rows1, rows2)
    ldss = (lds0, lds1, lds2)
    dsss = (dss0, dss1, dss2)
    gtss = (gts0, gts1, gts2)
    scss = (scs0, scs1, scs2)

    def ebase(ci):
        return s * npt + ci * K

    def fire_sn(ci, sl_):
        pltpu.async_copy(src_h.at[pl.ds(ebase(ci), K)], idxs[sl_], ldss[sl_])
        pltpu.async_copy(norm_h.at[pl.ds(ebase(ci), K)], nrms[sl_], ldss[sl_])

    def wait_sn(ci, sl_):
        pltpu.make_async_copy(src_h.at[pl.ds(ebase(ci), K)], idxs[sl_], ldss[sl_]).wait()
        pltpu.make_async_copy(norm_h.at[pl.ds(ebase(ci), K)], nrms[sl_], ldss[sl_]).wait()

    def fire_dst(ci, sl_):
        pltpu.async_copy(dst_h.at[pl.ds(ebase(ci), K)], dsts[sl_], dsss[sl_])

    def wait_dst(ci, sl_):
        pltpu.make_async_copy(dst_h.at[pl.ds(ebase(ci), K)], dsts[sl_], dsss[sl_]).wait()

    def adjust(sl_):
        for j in range(8):
            sl = pl.ds(j * 16, 16)
            idxs[sl_][sl] = idxs[sl_][sl] + coff

    def fire_gather(sl_):
        pltpu.async_copy(xw_h.at[idxs[sl_]], rows[sl_], gtss[sl_])

    def wait_gather(sl_):
        pltpu.make_async_copy(xw_h.at[idxs[sl_]], rows[sl_], gtss[sl_]).wait()

    def fire_scatter(sl_):
        pltpu.async_copy(rows[sl_], accS.at[dsts[sl_]], scss[sl_], add=True)

    def wait_scatter(sl_):
        pltpu.make_async_copy(rows[sl_], accS.at[dsts[sl_]], scss[sl_]).wait()

    def scale(sl_):
        @plsc.parallel_loop(0, K, unroll=8)
        def _body(k):
            nb = plsc.load_gather(nrms[sl_], [jnp.zeros((16,), i32) + k])
            for j in range(8):
                slc = pl.ds(j * 16, 16)
                rows[sl_][k, slc] = rows[sl_][k, slc] * nb

    zero16 = jnp.zeros((16,), f32)

    @plsc.parallel_loop(0, K, unroll=8)
    def _zrow(r):
        for j in range(8):
            rows0[r, pl.ds(j * 16, 16)] = zero16

    @pl.when(s <= NT - 2)
    def _zmain():
        for r in range(5):
            pltpu.sync_copy(rows0, accS.at[pl.ds(s * 640 + r * K, K)])

    @pl.when(s == NT - 1)
    def _zlast():
        for r in range(3):
            pltpu.sync_copy(rows0, accS.at[pl.ds(9600 + r * K, K)])
        pltpu.sync_copy(rows0.at[pl.ds(0, 32)], accS.at[pl.ds(9984, 32)])
    plsc.subcore_barrier()
    pltpu.sync_copy(b_h.at[pl.ds(c * 128, 128)], biasv)

    fire_sn(0, 0)
    fire_sn(1, 1)
    fire_sn(2, 2)
    fire_dst(0, 0)
    fire_dst(1, 1)
    wait_sn(0, 0)
    adjust(0)
    fire_gather(0)
    wait_sn(1, 1)
    adjust(1)
    fire_gather(1)

    def triple(q, carry):
        for sl_ in (0, 1, 2):
            ci = q * 3 + sl_
            s2 = (sl_ + 2) % 3

            @pl.when(ci >= 1)
            def _drain():
                wait_scatter(s2)

            @pl.when(ci <= nch - 3)
            def _stage():
                wait_sn(ci + 2, s2)
                adjust(s2)
                fire_dst(ci + 2, s2)
                fire_gather(s2)

            wait_gather(sl_)
            scale(sl_)
            wait_dst(ci, sl_)
            fire_scatter(sl_)

            @pl.when(ci <= nch - 4)
            def _prefetch():
                fire_sn(ci + 3, sl_)
        return carry

    lax.fori_loop(0, nch // 3, triple, 0)
    wait_scatter((nch - 1) % 3)
    plsc.subcore_barrier()

    bvecs = [biasv[pl.ds(j * 16, 16)] for j in range(8)]

    def do_piece(r0, ln):
        pltpu.sync_copy(accS.at[pl.ds(r0, ln)], rows0.at[pl.ds(0, ln)])

        @plsc.parallel_loop(0, ln, unroll=4)
        def _ep(r):
            for j in range(8):
                sl = pl.ds(j * 16, 16)
                rows0[r, sl] = jnp.maximum(rows0[r, sl] + bvecs[j], 0.0)

        pltpu.sync_copy(rows0.at[pl.ds(0, ln)], out_h.at[pl.ds(c * N + r0, ln)])

    base = s * 624
    for off, ln in ((0, K), (K, K), (2 * K, K), (3 * K, K), (4 * K, 112)):
        do_piece(base + off, ln)

    @pl.when(s == NT - 1)
    def _tail():
        do_piece(9984, 16)


_sc_params = pltpu.CompilerParams(needs_layout_passes=False)


def _make_prep(E2):
    return functools.partial(
        pl.kernel,
        out_type=jax.ShapeDtypeStruct((E2,), f32),
        mesh=_mesh,
        compiler_params=_sc_params,
        scratch_types=[
            pltpu.VMEM_SHARED((NPAD,), f32),
            pltpu.VMEM((K,), i32),
            pltpu.VMEM((K,), i32),
            pltpu.VMEM((K,), i32),
            pltpu.VMEM((K,), i32),
            pltpu.VMEM((K,), f32),
            pltpu.VMEM((K,), f32),
            pltpu.VMEM((640,), f32),
            pltpu.VMEM((N,), f32),
            pltpu.VMEM((K,), f32),
            pltpu.SemaphoreType.DMA,
            pltpu.SemaphoreType.DMA,
            pltpu.SemaphoreType.DMA,
            pltpu.SemaphoreType.DMA,
        ],
    )(_prep_body)


def _make_conv():
    return functools.partial(
        pl.kernel,
        out_type=jax.ShapeDtypeStruct((NC * N, 128), f32),
        mesh=_mesh,
        compiler_params=_sc_params,
        scratch_types=(
            [pltpu.VMEM_SHARED((NPAD, 128), f32)]
            + [pltpu.VMEM((K,), i32) for _ in range(6)]
            + [pltpu.VMEM((K,), f32) for _ in range(3)]
            + [pltpu.VMEM((K, 128), f32) for _ in range(3)]
            + [pltpu.VMEM((128,), f32)]
            + [pltpu.SemaphoreType.DMA for _ in range(12)]
        ),
    )(_conv_body)


def _mm1_body(x_ref, w_ref, o_ref):
    o_ref[...] = jnp.dot(x_ref[...], w_ref[...], preferred_element_type=f32)


def _mm2_body(xa_ref, xb_ref, w_ref, o_ref):
    o_ref[...] = (jnp.dot(xa_ref[...], w_ref[0:128], preferred_element_type=f32)
                  + jnp.dot(xb_ref[...], w_ref[128:256], preferred_element_type=f32))


def _lstm_body(x1a, x1b, x2a, x2b, x_ref, wt1, bb1, wt2, bb2, o_ref):
    g1 = (jnp.dot(x1a[...], wt1[0:128], preferred_element_type=f32)
          + jnp.dot(x1b[...], wt1[128:256], preferred_element_type=f32)
          + jnp.dot(x2a[...], wt1[256:384], preferred_element_type=f32)
          + jnp.dot(x2b[...], wt1[384:512], preferred_element_type=f32)) + bb1[...]
    gi = g1[:, 0:H]
    gg = g1[:, H:2 * H]
    go = g1[:, 2 * H:3 * H]
    cell = jax.nn.sigmoid(gi) * jnp.tanh(gg)
    h1 = jax.nn.sigmoid(go) * jnp.tanh(cell)
    g2 = jnp.dot(h1, wt2[...], preferred_element_type=f32) + bb2[...]
    gi2 = g2[:, 0:H]
    gg2 = g2[:, H:2 * H]
    go2 = g2[:, 2 * H:3 * H]
    cell2 = jax.nn.sigmoid(gi2) * jnp.tanh(gg2)
    h2 = jax.nn.sigmoid(go2) * jnp.tanh(cell2)
    o_ref[:, 0:H] = h1
    o_ref[:, H:2 * H] = h2
    o_ref[:, 2 * H:2 * H + C] = x_ref[...]


def kernel(X, edge_index, edge_weight, W1, b1, W2, b2,
           W_ih1, W_hh1, b_ih1, b_hh1, W_ih2, W_hh2, b_ih2, b_hh2):
    E = edge_weight.shape[0]
    loop = jnp.arange(N, dtype=edge_index.dtype)
    src = jnp.concatenate([edge_index[0], loop])
    dst = jnp.concatenate([edge_index[1], loop])
    w = jnp.concatenate([edge_weight, jnp.ones((N,), f32)])
    E2 = ((E + N + NT * NC * K * 3 - 1) // (NT * NC * K * 3)) * (NT * NC * K * 3)
    pad = E2 - (E + N)
    if pad:
        src = jnp.concatenate([src, jnp.zeros((pad,), src.dtype)])
        dst = jnp.concatenate([dst, jnp.zeros((pad,), dst.dtype)])
        w = jnp.concatenate([w, jnp.zeros((pad,), f32)])

    norm = _make_prep(E2)(src, dst, w)

    nblk = N // BM
    xw1 = pl.pallas_call(
        _mm1_body,
        grid=(nblk, NC),
        in_specs=[pl.BlockSpec((BM, C), lambda i, cc: (i, 0)),
                  pl.BlockSpec((C, 128), lambda i, cc: (0, cc))],
        out_specs=pl.BlockSpec((BM, 128), lambda i, cc: (cc * nblk + i, 0)),
        out_shape=jax.ShapeDtypeStruct((NC * N, 128), f32),
    )(X, W1)

    conv = _make_conv()
    x1 = conv(xw1, src, dst, norm, b1)

    xw2 = pl.pallas_call(
        _mm2_body,
        grid=(nblk, NC),
        in_specs=[pl.BlockSpec((BM, 128), lambda i, cc: (i, 0)),
                  pl.BlockSpec((BM, 128), lambda i, cc: (nblk + i, 0)),
                  pl.BlockSpec((H, 128), lambda i, cc: (0, cc))],
        out_specs=pl.BlockSpec((BM, 128), lambda i, cc: (cc * nblk + i, 0)),
        out_shape=jax.ShapeDtypeStruct((NC * N, 128), f32),
    )(x1, x1, W2)

    x2 = conv(xw2, src, dst, norm, b2)

    bb1 = b_ih1 + b_hh1
    Wt1 = jnp.concatenate([W_ih1[0:H], W_ih1[2 * H:4 * H]], axis=0).T
    bb1 = jnp.concatenate([bb1[0:H], bb1[2 * H:4 * H]]).reshape(1, 3 * H)
    bb2 = b_ih2 + b_hh2
    Wt2 = jnp.concatenate([W_ih2[0:H], W_ih2[2 * H:4 * H]], axis=0).T
    bb2 = jnp.concatenate([bb2[0:H], bb2[2 * H:4 * H]]).reshape(1, 3 * H)

    out = pl.pallas_call(
        _lstm_body,
        grid=(nblk,),
        in_specs=[pl.BlockSpec((BM, 128), lambda i: (i, 0)),
                  pl.BlockSpec((BM, 128), lambda i: (nblk + i, 0)),
                  pl.BlockSpec((BM, 128), lambda i: (i, 0)),
                  pl.BlockSpec((BM, 128), lambda i: (nblk + i, 0)),
                  pl.BlockSpec((BM, C), lambda i: (i, 0)),
                  pl.BlockSpec((2 * H, 3 * H), lambda i: (0, 0)),
                  pl.BlockSpec((1, 3 * H), lambda i: (0, 0)),
                  pl.BlockSpec((H, 3 * H), lambda i: (0, 0)),
                  pl.BlockSpec((1, 3 * H), lambda i: (0, 0)),
                  ],
        out_specs=pl.BlockSpec((BM, 2 * H + C), lambda i: (i, 0)),
        out_shape=jax.ShapeDtypeStruct((N, 2 * H + C), f32),
    )(x1, x1, x2, x2, X, Wt1, bb1, Wt2, bb2)
    return out
